# baseline (speedup 1.0000x reference)
import functools

import jax
import jax.numpy as jnp
from jax import lax
from jax.experimental import pallas as pl
from jax.experimental.pallas import tpu as pltpu
from jax.experimental.pallas import tpu_sc as plsc

VOCAB = 50257
N_POS = 2048
N_EMBD = 768
SEQ = 2048

_NC = 2
_NS = 16
_NW = _NC * _NS
_BPW = SEQ // _NW
_LANES = 16
_CHUNKS = N_EMBD // _LANES

_NCHUNK = 8
_RPC = _BPW // _NCHUNK

_mesh = plsc.VectorSubcoreMesh(core_axis_name="c", subcore_axis_name="s")


@functools.partial(
    pl.kernel,
    out_type=jax.ShapeDtypeStruct((SEQ, N_EMBD), jnp.float32),
    mesh=_mesh,
    scratch_types=[
        pltpu.VMEM((_BPW,), jnp.int32),
        pltpu.VMEM((_BPW, N_EMBD), jnp.float32),
        pltpu.VMEM((_BPW, N_EMBD), jnp.float32),
        pltpu.VMEM_SHARED((_NS * _BPW // 2, N_EMBD), jnp.float32),
        pltpu.SemaphoreType.DMA,
        pltpu.SemaphoreType.DMA,
        pltpu.SemaphoreType.DMA,
        pltpu.SemaphoreType.DMA,
        pltpu.SemaphoreType.DMA,
    ],
)
def _emb_lookup(wte_hbm, ids_hbm, wpe_hbm, out_hbm, ids_v, rows_v, wpe_v,
                wpe_sh, gsem, wsem, ssem, hsem, osem):
    sid = lax.axis_index("s")
    wid = sid * _NC + lax.axis_index("c")
    base = wid * _BPW
    lbase = sid * (_BPW // 2)
    _H = _NCHUNK // 2

    pltpu.sync_copy(ids_hbm.at[pl.ds(base, _BPW)], ids_v)

    def issue_direct(g, _):
        lo = g * _RPC
        pltpu.async_copy(wte_hbm.at[ids_v.at[pl.ds(lo, _RPC)]],
                         rows_v.at[pl.ds(lo, _RPC)], gsem)
        pltpu.async_copy(wpe_hbm.at[pl.ds(base + lo, _RPC)],
                         wpe_v.at[pl.ds(lo, _RPC)], wsem)
        return 0

    lax.fori_loop(0, _H, issue_direct, 0)

    def issue_staged(g, _):
        lo = g * _RPC
        pltpu.async_copy(wte_hbm.at[ids_v.at[pl.ds(lo, _RPC)]],
                         rows_v.at[pl.ds(lo, _RPC)], gsem)
        pltpu.async_copy(wpe_hbm.at[pl.ds(base + lo, _RPC)],
                         wpe_sh.at[pl.ds(lbase + lo - _H * _RPC, _RPC)], ssem)
        return 0

    lax.fori_loop(_H, _NCHUNK, issue_staged, 0)

    def hop(g, _):
        lo = g * _RPC
        pltpu.make_async_copy(wpe_hbm.at[pl.ds(0, _RPC)],
                              wpe_sh.at[pl.ds(0, _RPC)], ssem).wait()
        pltpu.async_copy(wpe_sh.at[pl.ds(lbase + lo - _H * _RPC, _RPC)],
                         wpe_v.at[pl.ds(lo, _RPC)], hsem)
        return 0

    lax.fori_loop(_H, _NCHUNK, hop, 0)

    def add_row(r, _):
        for c in range(_CHUNKS):
            sl = pl.ds(c * _LANES, _LANES)
            rows_v[r, sl] += wpe_v[r, sl]
        return 0

    def process_direct(g, _):
        lo = g * _RPC
        pltpu.make_async_copy(wte_hbm.at[pl.ds(0, _RPC)],
                              rows_v.at[pl.ds(lo, _RPC)], gsem).wait()
        pltpu.make_async_copy(wpe_hbm.at[pl.ds(0, _RPC)],
                              wpe_v.at[pl.ds(lo, _RPC)], wsem).wait()
        lax.fori_loop(lo, lo + _RPC, add_row, 0)
        pltpu.async_copy(rows_v.at[pl.ds(lo, _RPC)],
                         out_hbm.at[pl.ds(base + lo, _RPC)], osem)
        return 0

    lax.fori_loop(0, _H, process_direct, 0)

    def process_staged(g, _):
        lo = g * _RPC
        pltpu.make_async_copy(wte_hbm.at[pl.ds(0, _RPC)],
                              rows_v.at[pl.ds(lo, _RPC)], gsem).wait()
        pltpu.make_async_copy(wpe_sh.at[pl.ds(0, _RPC)],
                              wpe_v.at[pl.ds(lo, _RPC)], hsem).wait()
        lax.fori_loop(lo, lo + _RPC, add_row, 0)
        pltpu.async_copy(rows_v.at[pl.ds(lo, _RPC)],
                         out_hbm.at[pl.ds(base + lo, _RPC)], osem)
        return 0

    lax.fori_loop(_H, _NCHUNK, process_staged, 0)

    def drain(g, _):
        lo = g * _RPC
        pltpu.make_async_copy(rows_v.at[pl.ds(lo, _RPC)],
                              out_hbm.at[pl.ds(base + lo, _RPC)], osem).wait()
        return 0

    lax.fori_loop(0, _NCHUNK, drain, 0)


def kernel(input_ids, wte, wpe):
    ids = input_ids.astype(jnp.int32)
    out = _emb_lookup(wte, ids, wpe)
    return out[None, :, :]


# speedup vs baseline: 1.0263x; 1.0263x over previous
import functools

import jax
import jax.numpy as jnp
from jax import lax
from jax.experimental import pallas as pl
from jax.experimental.pallas import tpu as pltpu
from jax.experimental.pallas import tpu_sc as plsc

VOCAB = 50257
N_POS = 2048
N_EMBD = 768
SEQ = 2048

_NC = 2
_NS = 16
_NW = _NC * _NS
_BPW = SEQ // _NW
_LANES = 16
_CHUNKS = N_EMBD // _LANES

_NCHUNK = 4
_RPC = _BPW // _NCHUNK

_mesh = plsc.VectorSubcoreMesh(core_axis_name="c", subcore_axis_name="s")


@functools.partial(
    pl.kernel,
    out_type=jax.ShapeDtypeStruct((SEQ, N_EMBD), jnp.float32),
    mesh=_mesh,
    scratch_types=[
        pltpu.VMEM((_BPW,), jnp.int32),
        pltpu.VMEM((_BPW, N_EMBD), jnp.float32),
        pltpu.VMEM((_BPW, N_EMBD), jnp.float32),
        pltpu.SemaphoreType.DMA,
        pltpu.SemaphoreType.DMA,
        pltpu.SemaphoreType.DMA,
    ],
)
def _emb_lookup(wte_hbm, ids_hbm, wpe_hbm, out_hbm, ids_v, rows_v, wpe_v,
                gsem, wsem, osem):
    wid = lax.axis_index("s") * _NC + lax.axis_index("c")
    base = wid * _BPW

    pltpu.sync_copy(ids_hbm.at[pl.ds(base, _BPW)], ids_v)

    def issue(g, _):
        lo = g * _RPC
        pltpu.async_copy(wte_hbm.at[ids_v.at[pl.ds(lo, _RPC)]],
                         rows_v.at[pl.ds(lo, _RPC)], gsem)
        pltpu.async_copy(wpe_hbm.at[pl.ds(base + lo, _RPC)],
                         wpe_v.at[pl.ds(lo, _RPC)], wsem)
        return 0

    lax.fori_loop(0, _NCHUNK, issue, 0)

    def add_row(r, _):
        for c in range(_CHUNKS):
            sl = pl.ds(c * _LANES, _LANES)
            rows_v[r, sl] += wpe_v[r, sl]
        return 0

    def process(g, _):
        lo = g * _RPC
        pltpu.make_async_copy(wte_hbm.at[pl.ds(0, _RPC)],
                              rows_v.at[pl.ds(lo, _RPC)], gsem).wait()
        pltpu.make_async_copy(wpe_hbm.at[pl.ds(0, _RPC)],
                              wpe_v.at[pl.ds(lo, _RPC)], wsem).wait()
        lax.fori_loop(lo, lo + _RPC, add_row, 0)
        pltpu.async_copy(rows_v.at[pl.ds(lo, _RPC)],
                         out_hbm.at[pl.ds(base + lo, _RPC)], osem)
        return 0

    lax.fori_loop(0, _NCHUNK, process, 0)

    def drain(g, _):
        lo = g * _RPC
        pltpu.make_async_copy(rows_v.at[pl.ds(lo, _RPC)],
                              out_hbm.at[pl.ds(base + lo, _RPC)], osem).wait()
        return 0

    lax.fori_loop(0, _NCHUNK, drain, 0)


def kernel(input_ids, wte, wpe):
    ids = input_ids.astype(jnp.int32)
    out = _emb_lookup(wte, ids, wpe)
    return out[None, :, :]


# speedup vs baseline: 1.0395x; 1.0129x over previous
import functools

import jax
import jax.numpy as jnp
from jax import lax
from jax.experimental import pallas as pl
from jax.experimental.pallas import tpu as pltpu
from jax.experimental.pallas import tpu_sc as plsc

VOCAB = 50257
N_POS = 2048
N_EMBD = 768
SEQ = 2048

_NC = 2
_NS = 16
_NW = _NC * _NS
_BPW = SEQ // _NW
_LANES = 16
_CHUNKS = N_EMBD // _LANES

_NCHUNK = 8
_RPC = _BPW // _NCHUNK

_mesh = plsc.VectorSubcoreMesh(core_axis_name="c", subcore_axis_name="s")


@functools.partial(
    pl.kernel,
    out_type=jax.ShapeDtypeStruct((SEQ, N_EMBD), jnp.float32),
    mesh=_mesh,
    scratch_types=[
        pltpu.VMEM((_BPW,), jnp.int32),
        pltpu.VMEM((_BPW, N_EMBD), jnp.float32),
        pltpu.VMEM((_BPW, N_EMBD), jnp.float32),
        pltpu.SemaphoreType.DMA,
        pltpu.SemaphoreType.DMA,
        pltpu.SemaphoreType.DMA,
    ],
)
def _emb_lookup(wte_hbm, ids_hbm, wpe_hbm, out_hbm, ids_v, rows_v, wpe_v,
                gsem, wsem, osem):
    wid = lax.axis_index("s") * _NC + lax.axis_index("c")
    base = wid * _BPW

    pltpu.sync_copy(ids_hbm.at[pl.ds(base, _BPW)], ids_v)

    def issue(g, _):
        lo = g * _RPC
        pltpu.async_copy(wte_hbm.at[ids_v.at[pl.ds(lo, _RPC)]],
                         rows_v.at[pl.ds(lo, _RPC)], gsem)
        pltpu.async_copy(wpe_hbm.at[pl.ds(base + lo, _RPC)],
                         wpe_v.at[pl.ds(lo, _RPC)], wsem)
        return 0

    lax.fori_loop(0, _NCHUNK, issue, 0)

    def add_row(r, _):
        for c in range(_CHUNKS):
            sl = pl.ds(c * _LANES, _LANES)
            rows_v[r, sl] += wpe_v[r, sl]
        return 0

    def process(g, _):
        lo = g * _RPC
        pltpu.make_async_copy(wte_hbm.at[pl.ds(0, _RPC)],
                              rows_v.at[pl.ds(lo, _RPC)], gsem).wait()
        pltpu.make_async_copy(wpe_hbm.at[pl.ds(0, _RPC)],
                              wpe_v.at[pl.ds(lo, _RPC)], wsem).wait()
        lax.fori_loop(lo, lo + _RPC, add_row, 0)
        pltpu.async_copy(rows_v.at[pl.ds(lo, _RPC)],
                         out_hbm.at[pl.ds(base + lo, _RPC)], osem)
        return 0

    lax.fori_loop(0, _NCHUNK, process, 0)

    def drain(g, _):
        lo = g * _RPC
        pltpu.make_async_copy(rows_v.at[pl.ds(lo, _RPC)],
                              out_hbm.at[pl.ds(base + lo, _RPC)], osem).wait()
        return 0

    lax.fori_loop(0, _NCHUNK, drain, 0)


def kernel(input_ids, wte, wpe):
    ids = input_ids.astype(jnp.int32)
    out = _emb_lookup(wte, ids, wpe)
    return out[None, :, :]
